# baseline (speedup 1.0000x reference)
import functools

import jax
import jax.numpy as jnp
from jax import lax
from jax.experimental import pallas as pl
from jax.experimental.pallas import tpu as pltpu
from jax.experimental.pallas import tpu_sc as plsc

_N_PREDICT = 3
_H = 224
_W = 224
_F = _N_PREDICT * _H * _W
_NC = 2
_NS = 16


@functools.partial(jax.jit, static_argnums=(1,))
def _sc_broadcast(guess_flat, B):
    region = B * _F // (_NC * _NS)

    @functools.partial(
        pl.kernel,
        out_type=jax.ShapeDtypeStruct((B * _F,), jnp.float32),
        mesh=plsc.VectorSubcoreMesh(core_axis_name="c", subcore_axis_name="s"),
        scratch_types=[
            pltpu.VMEM((region,), jnp.float32),
        ],
    )
    def k(guess_hbm, out_hbm, buf):
        wid = lax.axis_index("s") * _NC + lax.axis_index("c")
        src_off = (wid * region) % _F
        pltpu.sync_copy(guess_hbm.at[pl.ds(src_off, region)], buf)
        pltpu.sync_copy(buf, out_hbm.at[pl.ds(wid * region, region)])

    return k(guess_flat)


def kernel(era5_land, guess):
    B = era5_land.shape[0]
    out = _sc_broadcast(guess.reshape(_F), B)
    return out.reshape(B, _N_PREDICT, _H, _W)


# speedup vs baseline: 2.5432x; 2.5432x over previous
import functools

import jax
import jax.numpy as jnp
from jax.experimental import pallas as pl

_N_PREDICT = 3
_H = 224
_W = 224


def _bcast_body(vec_ref, out_ref):
    out_ref[...] = vec_ref[...][None]


@functools.partial(jax.jit, static_argnums=(1,))
def _tc_broadcast(vec, B):
    return pl.pallas_call(
        _bcast_body,
        grid=(B,),
        in_specs=[pl.BlockSpec((_N_PREDICT, _H, _W), lambda b: (0, 0, 0))],
        out_specs=pl.BlockSpec((1, _N_PREDICT, _H, _W), lambda b: (b, 0, 0, 0)),
        out_shape=jax.ShapeDtypeStruct((B, _N_PREDICT, _H, _W), jnp.float32),
    )(vec)


def kernel(era5_land, guess):
    B = era5_land.shape[0]
    return _tc_broadcast(guess.reshape(_N_PREDICT, _H, _W), B)


# speedup vs baseline: 7.6111x; 2.9927x over previous
import functools

import jax
import jax.numpy as jnp
from jax.experimental import pallas as pl
from jax.experimental.pallas import tpu as pltpu

_N_PREDICT = 3
_H = 224
_W = 224
_R = _N_PREDICT * _H
_F = _R * _W


def _make_body(B):
    def body(vec_ref, out_hbm, scratch, sem):
        for r in range(_R):
            scratch[r, :] = vec_ref[0, pl.ds(r * _W, _W)]
        copies = [
            pltpu.async_copy(scratch, out_hbm.at[b], sem) for b in range(B)
        ]
        for c in copies:
            c.wait()

    return body


@functools.partial(jax.jit, static_argnums=(1,))
def _tc_broadcast(vec, B):
    out = pl.pallas_call(
        _make_body(B),
        in_specs=[pl.BlockSpec((1, _F), lambda: (0, 0))],
        out_specs=pl.BlockSpec(memory_space=pl.ANY),
        out_shape=jax.ShapeDtypeStruct((B, _R, _W), jnp.float32),
        scratch_shapes=[
            pltpu.VMEM((_R, _W), jnp.float32),
            pltpu.SemaphoreType.DMA,
        ],
    )(vec)
    return out.reshape(B, _N_PREDICT, _H, _W)


def kernel(era5_land, guess):
    B = era5_land.shape[0]
    return _tc_broadcast(guess, B)


# speedup vs baseline: 7.6347x; 1.0031x over previous
import functools

import jax
import jax.numpy as jnp
from jax.experimental import pallas as pl
from jax.experimental.pallas import tpu as pltpu

_N_PREDICT = 3
_H = 224
_W = 224
_R = _N_PREDICT * _H
_F = _R * _W


def _make_body(B):
    def body(vec_ref, out_hbm, scratch, sems):
        for r in range(_R):
            scratch[r, :] = vec_ref[0, pl.ds(r * _W, _W)]
        copies = [
            pltpu.async_copy(scratch, out_hbm.at[b], sems.at[b % 4])
            for b in range(B)
        ]
        for c in copies:
            c.wait()

    return body


@functools.partial(jax.jit, static_argnums=(1,))
def _tc_broadcast(vec, B):
    out = pl.pallas_call(
        _make_body(B),
        in_specs=[pl.BlockSpec((1, _F), lambda: (0, 0))],
        out_specs=pl.BlockSpec(memory_space=pl.ANY),
        out_shape=jax.ShapeDtypeStruct((B, _R, _W), jnp.float32),
        scratch_shapes=[
            pltpu.VMEM((_R, _W), jnp.float32),
            pltpu.SemaphoreType.DMA((4,)),
        ],
    )(vec)
    return out.reshape(B, _N_PREDICT, _H, _W)


def kernel(era5_land, guess):
    B = era5_land.shape[0]
    return _tc_broadcast(guess, B)
